# baseline (speedup 1.0000x reference)
import functools

import jax
import jax.numpy as jnp
from jax.experimental import pallas as pl
from jax.experimental.pallas import tpu as pltpu


def _pick_bt(B, C, HW, itemsize, target_bytes):
    per_b = C * HW * itemsize
    cap = max(1, target_bytes // per_b)
    bt = 1
    for d in range(1, min(B, cap) + 1):
        if B % d == 0:
            bt = d
    return bt


def _make_body(*, T, bt, S, P, Q, C, HW, inv_hw):
    Cq = C // Q

    def body(x_hbm, w1t_ref, b1_ref, w2t_ref, b2_ref, o_hbm,
             x_ring, o_ring, in_sems, out_sems):
        def issue_in(step, slot):
            for q in range(Q):
                pltpu.make_async_copy(
                    x_hbm.at[pl.ds(step * bt, bt), pl.ds(q * Cq, Cq), :],
                    x_ring.at[slot, :, pl.ds(q * Cq, Cq), :],
                    in_sems.at[slot],
                ).start()

        def wait_in(slot):
            pltpu.make_async_copy(
                x_ring.at[slot], x_ring.at[slot], in_sems.at[slot]
            ).wait()

        def issue_out(step, slot):
            for q in range(Q):
                pltpu.make_async_copy(
                    o_ring.at[slot, :, pl.ds(q * Cq, Cq), :],
                    o_hbm.at[pl.ds(step * bt, bt), pl.ds(q * Cq, Cq), :],
                    out_sems.at[slot],
                ).start(priority=1)

        def wait_out(slot):
            pltpu.make_async_copy(
                o_ring.at[slot], o_ring.at[slot], out_sems.at[slot]
            ).wait()

        for j in range(min(P, T)):
            issue_in(j, j % S)

        for i in range(T):
            if i + P < T:
                issue_in(i + P, (i + P) % S)
            cur = i % S
            wait_in(cur)
            x = x_ring[cur]
            pooled = jnp.sum(x, axis=-1, dtype=jnp.float32) * inv_hw
            h = jnp.dot(pooled, w1t_ref[...],
                        preferred_element_type=jnp.float32) + b1_ref[...]
            h = jnp.maximum(h, 0.0)
            s = jnp.dot(h, w2t_ref[...],
                        preferred_element_type=jnp.float32) + b2_ref[...]
            s = jax.nn.sigmoid(s)
            if i >= S:
                wait_out(cur)
            o_ring[cur] = x_ring[cur] * s[:, :, None]
            issue_out(i, cur)

        for j in range(max(0, T - S), T):
            wait_out(j % S)

    return body


@jax.jit
def kernel(x, w1, b1, w2, b2):
    B, C, H, W = x.shape
    Cr = w1.shape[0]
    HW = H * W
    xf = x.reshape(B, C, HW)
    w1t = w1.reshape(Cr, C).T
    w2t = w2.reshape(C, Cr).T
    b1r = b1.reshape(1, Cr)
    b2r = b2.reshape(1, C)

    itemsize = xf.dtype.itemsize
    bt = _pick_bt(B, C, HW, itemsize, 4 * 1024 * 1024)
    T = B // bt
    S = min(4, T)
    P = max(1, S - 1)
    Q = 4
    while C % Q != 0 and Q > 1:
        Q //= 2

    body = _make_body(T=T, bt=bt, S=S, P=P, Q=Q, C=C, HW=HW, inv_hw=1.0 / HW)

    out = pl.pallas_call(
        body,
        out_shape=jax.ShapeDtypeStruct((B, C, HW), xf.dtype),
        in_specs=[
            pl.BlockSpec(memory_space=pl.ANY),
            pl.BlockSpec((C, Cr), lambda: (0, 0)),
            pl.BlockSpec((1, Cr), lambda: (0, 0)),
            pl.BlockSpec((Cr, C), lambda: (0, 0)),
            pl.BlockSpec((1, C), lambda: (0, 0)),
        ],
        out_specs=pl.BlockSpec(memory_space=pl.ANY),
        scratch_shapes=[
            pltpu.VMEM((S, bt, C, HW), jnp.float32),
            pltpu.VMEM((S, bt, C, HW), jnp.float32),
            pltpu.SemaphoreType.DMA((S,)),
            pltpu.SemaphoreType.DMA((S,)),
        ],
        compiler_params=pltpu.CompilerParams(
            vmem_limit_bytes=56 * 1024 * 1024,
        ),
    )(xf, w1t, b1r, w2t, b2r)
    return out.reshape(B, C, H, W)
